# baseline (speedup 1.0000x reference)
import jax
import jax.numpy as jnp
from jax import lax
from jax.experimental import pallas as pl
from jax.experimental.pallas import tpu as pltpu
from jax.experimental.pallas import tpu_sc as plsc

_R = 10000

_NC = 2
_NS = 16
_NW = _NC * _NS
_L = 16


def _tc_body(nc_ref, t_ref, x_ref, out_ref):
    t = t_ref[0]
    c0 = nc_ref[0, 0]
    c1 = nc_ref[1, 0]
    c2 = nc_ref[2, 0]
    c3 = nc_ref[3, 0]
    f = jnp.where(t == 0, c0, jnp.where(t == 1, c1, jnp.where(t == 2, c2, c3)))
    f_col = jnp.transpose(f, (1, 0))
    out_ref[...] = x_ref[...] * f_col


def _tc_scale(node_features, t3d, norm_const):
    n, d = node_features.shape
    g = n // _R
    return pl.pallas_call(
        _tc_body,
        grid=(g,),
        in_specs=[
            pl.BlockSpec(memory_space=pltpu.SMEM),
            pl.BlockSpec((1, 1, _R), lambda i: (i, 0, 0)),
            pl.BlockSpec((_R, d), lambda i: (i, 0)),
        ],
        out_specs=pl.BlockSpec((_R, d), lambda i: (i, 0)),
        out_shape=jax.ShapeDtypeStruct((n, d), jnp.float32),
        compiler_params=pltpu.CompilerParams(
            dimension_semantics=("arbitrary",),
        ),
    )(norm_const, t3d, node_features)


def _sc_norm_factor(t_pad, nc_flat, chunk):
    np_len = t_pad.shape[0]
    mesh = plsc.VectorSubcoreMesh(core_axis_name="c", subcore_axis_name="s")

    def body(t_hbm, nc_hbm, out_hbm, t_v, f_v, nc_v):
        wid = lax.axis_index("s") * _NC + lax.axis_index("c")
        base = wid * chunk
        pltpu.sync_copy(nc_hbm, nc_v)
        pltpu.sync_copy(t_hbm.at[pl.ds(base, chunk)], t_v)
        ncv = nc_v[...]
        c0 = ncv[0]
        c1 = ncv[1]
        c2 = ncv[2]
        c3 = ncv[3]

        def step(j, carry):
            t = t_v[pl.ds(j * _L, _L)]
            f = jnp.where(
                t == 0, c0, jnp.where(t == 1, c1, jnp.where(t == 2, c2, c3))
            )
            f_v[pl.ds(j * _L, _L)] = f
            return carry

        lax.fori_loop(0, chunk // _L, step, 0, unroll=4)
        pltpu.sync_copy(f_v, out_hbm.at[pl.ds(base, chunk)])

    return pl.kernel(
        body,
        mesh=mesh,
        out_type=jax.ShapeDtypeStruct((np_len,), jnp.float32),
        scratch_types=[
            pltpu.VMEM((chunk,), jnp.int32),
            pltpu.VMEM((chunk,), jnp.float32),
            pltpu.VMEM((16,), jnp.float32),
        ],
    )(t_pad, nc_flat)


def kernel(node_features, atom_types, norm_const):
    n, d = node_features.shape
    g = n // _R
    t32 = atom_types.astype(jnp.int32)
    t3d = t32.reshape(g, 1, _R)

    chunk = -(-n // (_NW * _L)) * _L
    np_len = chunk * _NW
    t_pad = jnp.pad(t32, (0, np_len - n))
    nc16 = jnp.pad(norm_const.reshape(-1), (0, 16 - norm_const.size))
    nf_pad = _sc_norm_factor(t_pad, nc16, chunk)
    norm_factor = nf_pad[:n].reshape(n, 1)

    out_features = _tc_scale(node_features, t3d, norm_const)
    return out_features, norm_factor


# speedup vs baseline: 1.3506x; 1.3506x over previous
import jax
import jax.numpy as jnp
from jax.experimental import pallas as pl
from jax.experimental.pallas import tpu as pltpu

_R = 8192


def _body(nc_ref, t_ref, x_ref, out_ref, nf_ref):
    t = t_ref[...]
    c0 = nc_ref[0, 0]
    c1 = nc_ref[1, 0]
    c2 = nc_ref[2, 0]
    c3 = nc_ref[3, 0]
    f = jnp.where(t == 0, c0, jnp.where(t == 1, c1, jnp.where(t == 2, c2, c3)))
    nf_ref[...] = f
    f_col = f.reshape(_R, 1)
    out_ref[...] = x_ref[...] * f_col


def kernel(node_features, atom_types, norm_const):
    n, d = node_features.shape
    g = -(-n // _R)
    t32 = atom_types.astype(jnp.int32)
    out_features, nf = pl.pallas_call(
        _body,
        grid=(g,),
        in_specs=[
            pl.BlockSpec(memory_space=pltpu.SMEM),
            pl.BlockSpec((_R,), lambda i: (i,)),
            pl.BlockSpec((_R, d), lambda i: (i, 0)),
        ],
        out_specs=[
            pl.BlockSpec((_R, d), lambda i: (i, 0)),
            pl.BlockSpec((_R,), lambda i: (i,)),
        ],
        out_shape=[
            jax.ShapeDtypeStruct((n, d), jnp.float32),
            jax.ShapeDtypeStruct((n,), jnp.float32),
        ],
        compiler_params=pltpu.CompilerParams(
            dimension_semantics=("arbitrary",),
        ),
    )(norm_const, t32, node_features)
    return out_features, nf.reshape(n, 1)


# speedup vs baseline: 1.3629x; 1.0091x over previous
import jax
import jax.numpy as jnp
from jax.experimental import pallas as pl
from jax.experimental.pallas import tpu as pltpu

_R = 12288


def _body(nc_ref, t_ref, x_ref, out_ref, nf_ref):
    t = t_ref[...]
    c0 = nc_ref[0, 0]
    c1 = nc_ref[1, 0]
    c2 = nc_ref[2, 0]
    c3 = nc_ref[3, 0]
    f = jnp.where(t == 0, c0, jnp.where(t == 1, c1, jnp.where(t == 2, c2, c3)))
    nf_ref[...] = f
    f_col = f.reshape(_R, 1)
    out_ref[...] = x_ref[...] * f_col


def kernel(node_features, atom_types, norm_const):
    n, d = node_features.shape
    g = -(-n // _R)
    t32 = atom_types.astype(jnp.int32)
    out_features, nf = pl.pallas_call(
        _body,
        grid=(g,),
        in_specs=[
            pl.BlockSpec(memory_space=pltpu.SMEM),
            pl.BlockSpec((_R,), lambda i: (i,)),
            pl.BlockSpec((_R, d), lambda i: (i, 0)),
        ],
        out_specs=[
            pl.BlockSpec((_R, d), lambda i: (i, 0)),
            pl.BlockSpec((_R,), lambda i: (i,)),
        ],
        out_shape=[
            jax.ShapeDtypeStruct((n, d), jnp.float32),
            jax.ShapeDtypeStruct((n,), jnp.float32),
        ],
        compiler_params=pltpu.CompilerParams(
            dimension_semantics=("arbitrary",),
        ),
    )(norm_const, t32, node_features)
    return out_features, nf.reshape(n, 1)
